# baseline (speedup 1.0000x reference)
import math

import jax
import jax.numpy as jnp
from jax import lax
from jax.experimental import pallas as pl
from jax.experimental.pallas import tpu as pltpu
from jax.experimental.pallas import tpu_sc as plsc

EMB = 128
NUM_RADIAL = 6
CUTOFF = 5.0
N_NODES = 10000
N_EDGES = 320000
T_PAD = 96

_NC, _NS = 2, 16
_NW = _NC * _NS
_EDGES_PER_W = N_EDGES // _NW
_CHUNK = 2000


def _zgather_body(z_hbm, src_hbm, dst_hbm, zsrc_hbm, zdst_hbm,
                  z_v, si_v, di_v, so_v, do_v, sem_z, sem_s, sem_d):
    wid = lax.axis_index("s") * _NC + lax.axis_index("c")
    base = wid * _EDGES_PER_W
    cz = pltpu.make_async_copy(z_hbm, z_v, sem_z)
    cs = pltpu.make_async_copy(src_hbm.at[pl.ds(base, _EDGES_PER_W)], si_v,
                               sem_s)
    cd = pltpu.make_async_copy(dst_hbm.at[pl.ds(base, _EDGES_PER_W)], di_v,
                               sem_d)
    cz.start()
    cs.start()
    cd.start()
    cz.wait()
    cs.wait()
    cd.wait()

    @plsc.parallel_loop(0, _EDGES_PER_W, 16, unroll=8)
    def _grp(i):
        sl = pl.ds(i, 16)
        so_v[sl] = plsc.load_gather(z_v, [si_v[sl]])
        do_v[sl] = plsc.load_gather(z_v, [di_v[sl]])
    co = pltpu.make_async_copy(so_v, zsrc_hbm.at[pl.ds(base, _EDGES_PER_W)],
                               sem_s)
    cp = pltpu.make_async_copy(do_v, zdst_hbm.at[pl.ds(base, _EDGES_PER_W)],
                               sem_d)
    co.start()
    cp.start()
    co.wait()
    cp.wait()


def _sc_zgather(Z, src, dst):
    mesh = plsc.VectorSubcoreMesh(core_axis_name="c", subcore_axis_name="s")
    kern = pl.kernel(
        _zgather_body,
        mesh=mesh,
        compiler_params=pltpu.CompilerParams(needs_layout_passes=False),
        out_type=(
            jax.ShapeDtypeStruct((N_EDGES,), jnp.int32),
            jax.ShapeDtypeStruct((N_EDGES,), jnp.int32),
        ),
        scratch_types=[
            pltpu.VMEM((N_NODES,), jnp.int32),
            pltpu.VMEM((_EDGES_PER_W,), jnp.int32),
            pltpu.VMEM((_EDGES_PER_W,), jnp.int32),
            pltpu.VMEM((_EDGES_PER_W,), jnp.int32),
            pltpu.VMEM((_EDGES_PER_W,), jnp.int32),
            pltpu.SemaphoreType.DMA,
            pltpu.SemaphoreType.DMA,
            pltpu.SemaphoreType.DMA,
        ],
    )
    return kern(Z, src, dst)


def _prep_body(emb_ref, wd_ref, wr_ref, br_ref, bd_ref,
               t1_ref, t2_ref, wc_ref, bc_ref):
    emb = emb_ref[...]
    w1 = wd_ref[:, 0:EMB]
    w2 = wd_ref[:, EMB:2 * EMB]
    w3 = wd_ref[:, 2 * EMB:3 * EMB]
    dn = (((1,), (1,)), ((), ()))
    t1_ref[...] = lax.dot_general(emb, w1, dn,
                                  preferred_element_type=jnp.float32,
                                  precision=lax.Precision.HIGHEST
                                  ).astype(jnp.bfloat16)
    t2_ref[...] = lax.dot_general(emb, w2, dn,
                                  preferred_element_type=jnp.float32,
                                  precision=lax.Precision.HIGHEST
                                  ).astype(jnp.bfloat16)
    wc_ref[...] = lax.dot_general(wr_ref[...], w3, (((0,), (1,)), ((), ())),
                                  preferred_element_type=jnp.float32,
                                  precision=lax.Precision.HIGHEST)
    bc_ref[...] = lax.dot_general(br_ref[...], w3, (((1,), (1,)), ((), ())),
                                  preferred_element_type=jnp.float32,
                                  precision=lax.Precision.HIGHEST) + bd_ref[...]


def _tc_prep(emb_pad, W_dense, W_rbf, b_rbf2, b_dense2):
    return pl.pallas_call(
        _prep_body,
        out_shape=(
            jax.ShapeDtypeStruct((T_PAD, EMB), jnp.bfloat16),
            jax.ShapeDtypeStruct((T_PAD, EMB), jnp.bfloat16),
            jax.ShapeDtypeStruct((NUM_RADIAL, EMB), jnp.float32),
            jax.ShapeDtypeStruct((1, EMB), jnp.float32),
        ),
    )(emb_pad, W_dense, W_rbf, b_rbf2, b_dense2)


_NODE_BLK = 1000


def _h_body(z_ref, emb_ref, h_ref):
    z = z_ref[0]
    tt = lax.broadcasted_iota(jnp.int32, (T_PAD, _NODE_BLK), 0)
    oh = (tt == z).astype(jnp.float32)
    h_ref[...] = lax.dot_general(oh, emb_ref[...], (((0,), (0,)), ((), ())),
                                 preferred_element_type=jnp.float32,
                                 precision=lax.Precision.HIGHEST)


def _tc_h(Z3, emb_pad):
    grid = N_NODES // _NODE_BLK
    return pl.pallas_call(
        _h_body,
        grid=(grid,),
        in_specs=[
            pl.BlockSpec((1, 1, _NODE_BLK), lambda i: (i, 0, 0)),
            pl.BlockSpec((T_PAD, EMB), lambda i: (0, 0)),
        ],
        out_specs=pl.BlockSpec((_NODE_BLK, EMB), lambda i: (i, 0)),
        out_shape=jax.ShapeDtypeStruct((N_NODES, EMB), jnp.float32),
    )(Z3, emb_pad)


_EDGE_BLK = 2560
_ENV_P = 6


def _edge_body(zs_ref, zd_ref, rbfT_ref, d_ref, t1_ref, t2_ref, wc_ref, bc_ref,
               m_ref, envT_ref):
    tt = lax.broadcasted_iota(jnp.int32, (T_PAD, _EDGE_BLK), 0)
    dnT = (((0,), (0,)), ((), ()))
    oh_s = (tt == zs_ref[0]).astype(jnp.bfloat16)
    acc = lax.dot_general(oh_s, t1_ref[...], dnT,
                          preferred_element_type=jnp.float32)
    oh_d = (tt == zd_ref[0]).astype(jnp.bfloat16)
    acc = acc + lax.dot_general(oh_d, t2_ref[...], dnT,
                                preferred_element_type=jnp.float32)
    acc = acc + lax.dot_general(rbfT_ref[...], wc_ref[...], dnT,
                                preferred_element_type=jnp.float32)
    m_ref[...] = acc + bc_ref[...]

    x = d_ref[0] * (1.0 / CUTOFF)
    inv = 1.0 / x
    p = _ENV_P
    a = -(p + 1) * (p + 2) / 2.0
    b = p * (p + 2)
    c = -p * (p + 1) / 2.0
    x2 = x * x
    x4 = x2 * x2
    xp0 = x4 * x
    env = inv + xp0 * (a + x * (b + c * x))
    coef = env * inv * math.sqrt(2.0 / CUTOFF)
    theta = x * math.pi
    s1 = jnp.sin(theta)
    two_c = 2.0 * jnp.cos(theta)
    rows = [s1, two_c * s1]
    for _ in range(NUM_RADIAL - 2):
        rows.append(two_c * rows[-1] - rows[-2])
    envT_ref[...] = coef * jnp.concatenate(rows, axis=0)


def _tc_edge(zsrc3, zdst3, rbfT, d3, t1, t2, wc, bc):
    grid = N_EDGES // _EDGE_BLK
    return pl.pallas_call(
        _edge_body,
        grid=(grid,),
        in_specs=[
            pl.BlockSpec((1, 1, _EDGE_BLK), lambda i: (i, 0, 0)),
            pl.BlockSpec((1, 1, _EDGE_BLK), lambda i: (i, 0, 0)),
            pl.BlockSpec((NUM_RADIAL, _EDGE_BLK), lambda i: (0, i)),
            pl.BlockSpec((1, 1, _EDGE_BLK), lambda i: (i, 0, 0)),
            pl.BlockSpec((T_PAD, EMB), lambda i: (0, 0)),
            pl.BlockSpec((T_PAD, EMB), lambda i: (0, 0)),
            pl.BlockSpec((NUM_RADIAL, EMB), lambda i: (0, 0)),
            pl.BlockSpec((1, EMB), lambda i: (0, 0)),
        ],
        out_specs=(
            pl.BlockSpec((_EDGE_BLK, EMB), lambda i: (i, 0)),
            pl.BlockSpec((NUM_RADIAL, _EDGE_BLK), lambda i: (0, i)),
        ),
        out_shape=(
            jax.ShapeDtypeStruct((N_EDGES, EMB), jnp.float32),
            jax.ShapeDtypeStruct((NUM_RADIAL, N_EDGES), jnp.float32),
        ),
    )(zsrc3, zdst3, rbfT, d3, t1, t2, wc, bc)


def kernel(Z, edge_index, rbf, d, emb_table, W_rbf, b_rbf, W_dense, b_dense):
    Z = Z.astype(jnp.int32)
    src = edge_index[0].astype(jnp.int32)
    dst = edge_index[1].astype(jnp.int32)

    emb_pad = jnp.zeros((T_PAD, EMB), jnp.float32).at[:emb_table.shape[0]].set(
        emb_table)
    b_rbf2 = b_rbf.reshape(1, EMB)
    b_dense2 = b_dense.reshape(1, EMB)

    zsrc, zdst = _sc_zgather(Z, src, dst)

    t1, t2, wc, bc = _tc_prep(emb_pad, W_dense, W_rbf, b_rbf2, b_dense2)
    h = _tc_h(Z.reshape(N_NODES // _NODE_BLK, 1, _NODE_BLK), emb_pad)
    g = N_EDGES // _EDGE_BLK
    m, rbf_envT = _tc_edge(zsrc.reshape(g, 1, _EDGE_BLK),
                           zdst.reshape(g, 1, _EDGE_BLK),
                           rbf.T, d.reshape(g, 1, _EDGE_BLK), t1, t2, wc, bc)
    return h, m, rbf_envT.T


# speedup vs baseline: 1.1832x; 1.1832x over previous
import math

import jax
import jax.numpy as jnp
from jax import lax
from jax.experimental import pallas as pl
from jax.experimental.pallas import tpu as pltpu
from jax.experimental.pallas import tpu_sc as plsc

EMB = 128
NUM_RADIAL = 6
CUTOFF = 5.0
N_NODES = 10000
N_EDGES = 320000
T_PAD = 128

_NC, _NS = 2, 16
_NW = _NC * _NS
_EDGES_PER_W = N_EDGES // _NW
_CHUNK = 2000


def _zgather_body(z_hbm, src_hbm, dst_hbm, zsrc_hbm, zdst_hbm,
                  z_v, si_v, di_v, so_v, do_v, sem_z, sem_s, sem_d):
    wid = lax.axis_index("s") * _NC + lax.axis_index("c")
    base = wid * _EDGES_PER_W
    cz = pltpu.make_async_copy(z_hbm, z_v, sem_z)
    cs = pltpu.make_async_copy(src_hbm.at[pl.ds(base, _EDGES_PER_W)], si_v,
                               sem_s)
    cd = pltpu.make_async_copy(dst_hbm.at[pl.ds(base, _EDGES_PER_W)], di_v,
                               sem_d)
    cz.start()
    cs.start()
    cd.start()
    cz.wait()
    cs.wait()
    cd.wait()

    @plsc.parallel_loop(0, _EDGES_PER_W, 16, unroll=8)
    def _grp(i):
        sl = pl.ds(i, 16)
        so_v[sl] = plsc.load_gather(z_v, [si_v[sl]])
        do_v[sl] = plsc.load_gather(z_v, [di_v[sl]])
    co = pltpu.make_async_copy(so_v, zsrc_hbm.at[pl.ds(base, _EDGES_PER_W)],
                               sem_s)
    cp = pltpu.make_async_copy(do_v, zdst_hbm.at[pl.ds(base, _EDGES_PER_W)],
                               sem_d)
    co.start()
    cp.start()
    co.wait()
    cp.wait()


def _sc_zgather(Z, src, dst):
    mesh = plsc.VectorSubcoreMesh(core_axis_name="c", subcore_axis_name="s")
    kern = pl.kernel(
        _zgather_body,
        mesh=mesh,
        compiler_params=pltpu.CompilerParams(needs_layout_passes=False),
        out_type=(
            jax.ShapeDtypeStruct((N_EDGES,), jnp.int32),
            jax.ShapeDtypeStruct((N_EDGES,), jnp.int32),
        ),
        scratch_types=[
            pltpu.VMEM((N_NODES,), jnp.int32),
            pltpu.VMEM((_EDGES_PER_W,), jnp.int32),
            pltpu.VMEM((_EDGES_PER_W,), jnp.int32),
            pltpu.VMEM((_EDGES_PER_W,), jnp.int32),
            pltpu.VMEM((_EDGES_PER_W,), jnp.int32),
            pltpu.SemaphoreType.DMA,
            pltpu.SemaphoreType.DMA,
            pltpu.SemaphoreType.DMA,
        ],
    )
    return kern(Z, src, dst)


def _prep_body(emb_ref, wd_ref, wr_ref, br_ref, bd_ref,
               t1_ref, t2_ref, wc_ref, bc_ref):
    emb = emb_ref[...]
    w1 = wd_ref[:, 0:EMB]
    w2 = wd_ref[:, EMB:2 * EMB]
    w3 = wd_ref[:, 2 * EMB:3 * EMB]
    dn = (((1,), (1,)), ((), ()))
    t1_ref[...] = lax.dot_general(emb, w1, dn,
                                  preferred_element_type=jnp.float32,
                                  precision=lax.Precision.HIGHEST
                                  ).astype(jnp.bfloat16)
    t2_ref[...] = lax.dot_general(emb, w2, dn,
                                  preferred_element_type=jnp.float32,
                                  precision=lax.Precision.HIGHEST
                                  ).astype(jnp.bfloat16)
    wc_ref[...] = lax.dot_general(wr_ref[...], w3, (((0,), (1,)), ((), ())),
                                  preferred_element_type=jnp.float32,
                                  precision=lax.Precision.HIGHEST)
    bc_ref[...] = lax.dot_general(br_ref[...], w3, (((1,), (1,)), ((), ())),
                                  preferred_element_type=jnp.float32,
                                  precision=lax.Precision.HIGHEST) + bd_ref[...]


def _tc_prep(emb_pad, W_dense, W_rbf, b_rbf2, b_dense2):
    return pl.pallas_call(
        _prep_body,
        out_shape=(
            jax.ShapeDtypeStruct((T_PAD, EMB), jnp.bfloat16),
            jax.ShapeDtypeStruct((T_PAD, EMB), jnp.bfloat16),
            jax.ShapeDtypeStruct((NUM_RADIAL, EMB), jnp.float32),
            jax.ShapeDtypeStruct((1, EMB), jnp.float32),
        ),
    )(emb_pad, W_dense, W_rbf, b_rbf2, b_dense2)


_NODE_BLK = 1000


def _h_body(z_ref, emb_ref, h_ref):
    z = z_ref[0]
    tt = lax.broadcasted_iota(jnp.int32, (T_PAD, _NODE_BLK), 0)
    oh = (tt == z).astype(jnp.float32)
    h_ref[...] = lax.dot_general(oh, emb_ref[...], (((0,), (0,)), ((), ())),
                                 preferred_element_type=jnp.float32,
                                 precision=lax.Precision.HIGHEST)


def _tc_h(Z3, emb_pad):
    grid = N_NODES // _NODE_BLK
    return pl.pallas_call(
        _h_body,
        grid=(grid,),
        in_specs=[
            pl.BlockSpec((1, 1, _NODE_BLK), lambda i: (i, 0, 0)),
            pl.BlockSpec((T_PAD, EMB), lambda i: (0, 0)),
        ],
        out_specs=pl.BlockSpec((_NODE_BLK, EMB), lambda i: (i, 0)),
        out_shape=jax.ShapeDtypeStruct((N_NODES, EMB), jnp.float32),
    )(Z3, emb_pad)


_EDGE_BLK = 6400
_ENV_P = 6


def _edge_body(zs_ref, zd_ref, rbfT_ref, d_ref, t1_ref, t2_ref, wc_ref, bc_ref,
               m_ref, envT_ref):
    tt = lax.broadcasted_iota(jnp.int32, (T_PAD, _EDGE_BLK), 0)
    dnT = (((0,), (0,)), ((), ()))
    oh_s = (tt == zs_ref[0]).astype(jnp.bfloat16)
    acc = lax.dot_general(oh_s, t1_ref[...], dnT,
                          preferred_element_type=jnp.float32)
    oh_d = (tt == zd_ref[0]).astype(jnp.bfloat16)
    acc = acc + lax.dot_general(oh_d, t2_ref[...], dnT,
                                preferred_element_type=jnp.float32)
    acc = acc + lax.dot_general(rbfT_ref[...], wc_ref[...], dnT,
                                preferred_element_type=jnp.float32)
    m_ref[...] = acc + bc_ref[...]

    x = d_ref[0] * (1.0 / CUTOFF)
    inv = 1.0 / x
    p = _ENV_P
    a = -(p + 1) * (p + 2) / 2.0
    b = p * (p + 2)
    c = -p * (p + 1) / 2.0
    x2 = x * x
    x4 = x2 * x2
    xp0 = x4 * x
    env = inv + xp0 * (a + x * (b + c * x))
    coef = env * inv * math.sqrt(2.0 / CUTOFF)
    theta = x * math.pi
    s1 = jnp.sin(theta)
    two_c = 2.0 * jnp.cos(theta)
    rows = [s1, two_c * s1]
    for _ in range(NUM_RADIAL - 2):
        rows.append(two_c * rows[-1] - rows[-2])
    envT_ref[...] = coef * jnp.concatenate(rows, axis=0)


def _tc_edge(zsrc3, zdst3, rbfT, d3, t1, t2, wc, bc):
    grid = N_EDGES // _EDGE_BLK
    return pl.pallas_call(
        _edge_body,
        grid=(grid,),
        in_specs=[
            pl.BlockSpec((1, 1, _EDGE_BLK), lambda i: (i, 0, 0)),
            pl.BlockSpec((1, 1, _EDGE_BLK), lambda i: (i, 0, 0)),
            pl.BlockSpec((NUM_RADIAL, _EDGE_BLK), lambda i: (0, i)),
            pl.BlockSpec((1, 1, _EDGE_BLK), lambda i: (i, 0, 0)),
            pl.BlockSpec((T_PAD, EMB), lambda i: (0, 0)),
            pl.BlockSpec((T_PAD, EMB), lambda i: (0, 0)),
            pl.BlockSpec((NUM_RADIAL, EMB), lambda i: (0, 0)),
            pl.BlockSpec((1, EMB), lambda i: (0, 0)),
        ],
        out_specs=(
            pl.BlockSpec((_EDGE_BLK, EMB), lambda i: (i, 0)),
            pl.BlockSpec((NUM_RADIAL, _EDGE_BLK), lambda i: (0, i)),
        ),
        out_shape=(
            jax.ShapeDtypeStruct((N_EDGES, EMB), jnp.float32),
            jax.ShapeDtypeStruct((NUM_RADIAL, N_EDGES), jnp.float32),
        ),
    )(zsrc3, zdst3, rbfT, d3, t1, t2, wc, bc)


def kernel(Z, edge_index, rbf, d, emb_table, W_rbf, b_rbf, W_dense, b_dense):
    Z = Z.astype(jnp.int32)
    src = edge_index[0].astype(jnp.int32)
    dst = edge_index[1].astype(jnp.int32)

    emb_pad = jnp.zeros((T_PAD, EMB), jnp.float32).at[:emb_table.shape[0]].set(
        emb_table)
    b_rbf2 = b_rbf.reshape(1, EMB)
    b_dense2 = b_dense.reshape(1, EMB)

    zsrc, zdst = _sc_zgather(Z, src, dst)

    t1, t2, wc, bc = _tc_prep(emb_pad, W_dense, W_rbf, b_rbf2, b_dense2)
    h = _tc_h(Z.reshape(N_NODES // _NODE_BLK, 1, _NODE_BLK), emb_pad)
    g = N_EDGES // _EDGE_BLK
    m, rbf_envT = _tc_edge(zsrc.reshape(g, 1, _EDGE_BLK),
                           zdst.reshape(g, 1, _EDGE_BLK),
                           rbf.T, d.reshape(g, 1, _EDGE_BLK), t1, t2, wc, bc)
    return h, m, rbf_envT.T


# speedup vs baseline: 1.2215x; 1.0323x over previous
import math

import jax
import jax.numpy as jnp
from jax import lax
from jax.experimental import pallas as pl
from jax.experimental.pallas import tpu as pltpu
from jax.experimental.pallas import tpu_sc as plsc

EMB = 128
NUM_RADIAL = 6
CUTOFF = 5.0
N_NODES = 10000
N_EDGES = 320000
T_PAD = 128

_NC, _NS = 2, 16
_NW = _NC * _NS
_EDGES_PER_W = N_EDGES // _NW
_CHUNK = 2000


def _zgather_body(z_hbm, src_hbm, dst_hbm, zsrc_hbm, zdst_hbm,
                  z_v, si_v, di_v, so_v, do_v, sem_z, sem_s, sem_d):
    wid = lax.axis_index("s") * _NC + lax.axis_index("c")
    base = wid * _EDGES_PER_W
    cz = pltpu.make_async_copy(z_hbm, z_v, sem_z)
    cs = pltpu.make_async_copy(src_hbm.at[pl.ds(base, _EDGES_PER_W)], si_v,
                               sem_s)
    cd = pltpu.make_async_copy(dst_hbm.at[pl.ds(base, _EDGES_PER_W)], di_v,
                               sem_d)
    cz.start()
    cs.start()
    cd.start()
    cz.wait()
    cs.wait()
    cd.wait()

    @plsc.parallel_loop(0, _EDGES_PER_W, 16, unroll=8)
    def _grp(i):
        sl = pl.ds(i, 16)
        so_v[sl] = plsc.load_gather(z_v, [si_v[sl]])
        do_v[sl] = plsc.load_gather(z_v, [di_v[sl]])
    co = pltpu.make_async_copy(so_v, zsrc_hbm.at[pl.ds(base, _EDGES_PER_W)],
                               sem_s)
    cp = pltpu.make_async_copy(do_v, zdst_hbm.at[pl.ds(base, _EDGES_PER_W)],
                               sem_d)
    co.start()
    cp.start()
    co.wait()
    cp.wait()


def _sc_zgather(Z, src, dst):
    mesh = plsc.VectorSubcoreMesh(core_axis_name="c", subcore_axis_name="s")
    kern = pl.kernel(
        _zgather_body,
        mesh=mesh,
        compiler_params=pltpu.CompilerParams(needs_layout_passes=False),
        out_type=(
            jax.ShapeDtypeStruct((N_EDGES,), jnp.int32),
            jax.ShapeDtypeStruct((N_EDGES,), jnp.int32),
        ),
        scratch_types=[
            pltpu.VMEM((N_NODES,), jnp.int32),
            pltpu.VMEM((_EDGES_PER_W,), jnp.int32),
            pltpu.VMEM((_EDGES_PER_W,), jnp.int32),
            pltpu.VMEM((_EDGES_PER_W,), jnp.int32),
            pltpu.VMEM((_EDGES_PER_W,), jnp.int32),
            pltpu.SemaphoreType.DMA,
            pltpu.SemaphoreType.DMA,
            pltpu.SemaphoreType.DMA,
        ],
    )
    return kern(Z, src, dst)


def _prep_body(emb_ref, wd_ref, wr_ref, br_ref, bd_ref,
               t1_ref, t2_ref, wc_ref, bc_ref):
    emb = emb_ref[...]
    w1 = wd_ref[:, 0:EMB]
    w2 = wd_ref[:, EMB:2 * EMB]
    w3 = wd_ref[:, 2 * EMB:3 * EMB]
    dn = (((1,), (1,)), ((), ()))
    t1_ref[...] = lax.dot_general(emb, w1, dn,
                                  preferred_element_type=jnp.float32,
                                  precision=lax.Precision.HIGHEST
                                  ).astype(jnp.bfloat16)
    t2_ref[...] = lax.dot_general(emb, w2, dn,
                                  preferred_element_type=jnp.float32,
                                  precision=lax.Precision.HIGHEST
                                  ).astype(jnp.bfloat16)
    wc_ref[...] = lax.dot_general(wr_ref[...], w3, (((0,), (1,)), ((), ())),
                                  preferred_element_type=jnp.float32,
                                  precision=lax.Precision.HIGHEST)
    bc_ref[...] = lax.dot_general(br_ref[...], w3, (((1,), (1,)), ((), ())),
                                  preferred_element_type=jnp.float32,
                                  precision=lax.Precision.HIGHEST) + bd_ref[...]


def _tc_prep(emb_pad, W_dense, W_rbf, b_rbf2, b_dense2):
    return pl.pallas_call(
        _prep_body,
        out_shape=(
            jax.ShapeDtypeStruct((T_PAD, EMB), jnp.bfloat16),
            jax.ShapeDtypeStruct((T_PAD, EMB), jnp.bfloat16),
            jax.ShapeDtypeStruct((NUM_RADIAL, EMB), jnp.float32),
            jax.ShapeDtypeStruct((1, EMB), jnp.float32),
        ),
    )(emb_pad, W_dense, W_rbf, b_rbf2, b_dense2)


_NODE_BLK = 1000


def _h_body(z_ref, emb_ref, h_ref):
    z = z_ref[0]
    tt = lax.broadcasted_iota(jnp.int32, (T_PAD, _NODE_BLK), 0)
    oh = (tt == z).astype(jnp.float32)
    h_ref[...] = lax.dot_general(oh, emb_ref[...], (((0,), (0,)), ((), ())),
                                 preferred_element_type=jnp.float32,
                                 precision=lax.Precision.HIGHEST)


def _tc_h(Z3, emb_pad):
    grid = N_NODES // _NODE_BLK
    return pl.pallas_call(
        _h_body,
        grid=(grid,),
        in_specs=[
            pl.BlockSpec((1, 1, _NODE_BLK), lambda i: (i, 0, 0)),
            pl.BlockSpec((T_PAD, EMB), lambda i: (0, 0)),
        ],
        out_specs=pl.BlockSpec((_NODE_BLK, EMB), lambda i: (i, 0)),
        out_shape=jax.ShapeDtypeStruct((N_NODES, EMB), jnp.float32),
    )(Z3, emb_pad)


_EDGE_BLK = 12800
_ENV_P = 6


def _edge_body(zs_ref, zd_ref, rbfT_ref, d_ref, t1_ref, t2_ref, wc_ref, bc_ref,
               m_ref, envT_ref):
    tt = lax.broadcasted_iota(jnp.int32, (T_PAD, _EDGE_BLK), 0)
    dnT = (((0,), (0,)), ((), ()))
    oh_s = (tt == zs_ref[0]).astype(jnp.bfloat16)
    acc = lax.dot_general(oh_s, t1_ref[...], dnT,
                          preferred_element_type=jnp.float32)
    oh_d = (tt == zd_ref[0]).astype(jnp.bfloat16)
    acc = acc + lax.dot_general(oh_d, t2_ref[...], dnT,
                                preferred_element_type=jnp.float32)
    acc = acc + lax.dot_general(rbfT_ref[...], wc_ref[...], dnT,
                                preferred_element_type=jnp.float32)
    m_ref[...] = acc + bc_ref[...]

    x = d_ref[0] * (1.0 / CUTOFF)
    inv = 1.0 / x
    p = _ENV_P
    a = -(p + 1) * (p + 2) / 2.0
    b = p * (p + 2)
    c = -p * (p + 1) / 2.0
    x2 = x * x
    x4 = x2 * x2
    xp0 = x4 * x
    env = inv + xp0 * (a + x * (b + c * x))
    coef = env * inv * math.sqrt(2.0 / CUTOFF)
    theta = x * math.pi
    s1 = jnp.sin(theta)
    two_c = 2.0 * jnp.cos(theta)
    rows = [s1, two_c * s1]
    for _ in range(NUM_RADIAL - 2):
        rows.append(two_c * rows[-1] - rows[-2])
    envT_ref[...] = coef * jnp.concatenate(rows, axis=0)


def _tc_edge(zsrc3, zdst3, rbfT, d3, t1, t2, wc, bc):
    grid = N_EDGES // _EDGE_BLK
    return pl.pallas_call(
        _edge_body,
        grid=(grid,),
        in_specs=[
            pl.BlockSpec((1, 1, _EDGE_BLK), lambda i: (i, 0, 0)),
            pl.BlockSpec((1, 1, _EDGE_BLK), lambda i: (i, 0, 0)),
            pl.BlockSpec((NUM_RADIAL, _EDGE_BLK), lambda i: (0, i)),
            pl.BlockSpec((1, 1, _EDGE_BLK), lambda i: (i, 0, 0)),
            pl.BlockSpec((T_PAD, EMB), lambda i: (0, 0)),
            pl.BlockSpec((T_PAD, EMB), lambda i: (0, 0)),
            pl.BlockSpec((NUM_RADIAL, EMB), lambda i: (0, 0)),
            pl.BlockSpec((1, EMB), lambda i: (0, 0)),
        ],
        out_specs=(
            pl.BlockSpec((_EDGE_BLK, EMB), lambda i: (i, 0)),
            pl.BlockSpec((NUM_RADIAL, _EDGE_BLK), lambda i: (0, i)),
        ),
        out_shape=(
            jax.ShapeDtypeStruct((N_EDGES, EMB), jnp.float32),
            jax.ShapeDtypeStruct((NUM_RADIAL, N_EDGES), jnp.float32),
        ),
    )(zsrc3, zdst3, rbfT, d3, t1, t2, wc, bc)


def kernel(Z, edge_index, rbf, d, emb_table, W_rbf, b_rbf, W_dense, b_dense):
    Z = Z.astype(jnp.int32)
    src = edge_index[0].astype(jnp.int32)
    dst = edge_index[1].astype(jnp.int32)

    emb_pad = jnp.zeros((T_PAD, EMB), jnp.float32).at[:emb_table.shape[0]].set(
        emb_table)
    b_rbf2 = b_rbf.reshape(1, EMB)
    b_dense2 = b_dense.reshape(1, EMB)

    zsrc, zdst = _sc_zgather(Z, src, dst)

    t1, t2, wc, bc = _tc_prep(emb_pad, W_dense, W_rbf, b_rbf2, b_dense2)
    h = _tc_h(Z.reshape(N_NODES // _NODE_BLK, 1, _NODE_BLK), emb_pad)
    g = N_EDGES // _EDGE_BLK
    m, rbf_envT = _tc_edge(zsrc.reshape(g, 1, _EDGE_BLK),
                           zdst.reshape(g, 1, _EDGE_BLK),
                           rbf.T, d.reshape(g, 1, _EDGE_BLK), t1, t2, wc, bc)
    return h, m, rbf_envT.T
